# baseline (speedup 1.0000x reference)
import functools

import jax
import jax.numpy as jnp
from jax import lax
from jax.experimental import pallas as pl
from jax.experimental.pallas import tpu as pltpu
from jax.experimental.pallas import tpu_sc as plsc

_NC = 2
_NS = 16
_NW = _NC * _NS
_LANES = 16
_CHUNK = 128


@functools.cache
def _build(batch: int):
    P = batch // _NW
    NCH = P // _CHUNK
    CPR = _CHUNK // _LANES

    mesh = plsc.VectorSubcoreMesh(
        core_axis_name="c", subcore_axis_name="s",
        num_cores=_NC, num_subcores=_NS)

    @functools.partial(
        pl.kernel,
        out_type=jax.ShapeDtypeStruct((batch,), jnp.float32),
        mesh=mesh,
        scratch_types=[
            pltpu.VMEM((3 * P,), jnp.float32),
            pltpu.VMEM((2 * NCH, _CHUNK), jnp.int32),
            pltpu.VMEM((2 * NCH, _CHUNK), jnp.float32),
            pltpu.VMEM((P,), jnp.float32),
            pltpu.SemaphoreType.DMA,
            pltpu.SemaphoreType.DMA((NCH,)),
        ],
    )
    def body(packed_hbm, zetas_hbm, out_hbm, xin, g, z, lv, isem, gsem):
        wid = lax.axis_index("s") * _NC + lax.axis_index("c")
        base = wid * P
        pltpu.async_copy(
            packed_hbm.at[pl.ds(3 * base, 3 * P)], xin, isem).wait()

        gathers = []
        for r in range(NCH):
            for k in range(CPR):
                off = (r * CPR + k) * _LANES
                csl = pl.ds(k * _LANES, _LANES)
                fa = jnp.maximum(xin[pl.ds(off, _LANES)] - 1.0, 0.0)
                fb = jnp.maximum(xin[pl.ds(P + off, _LANES)] - 1.0, 0.0)
                g[r, csl] = fa.astype(jnp.int32)
                g[NCH + r, csl] = fb.astype(jnp.int32)
            gathers.append((
                pltpu.async_copy(zetas_hbm.at[g.at[r]], z.at[r], gsem.at[r]),
                pltpu.async_copy(zetas_hbm.at[g.at[NCH + r]], z.at[NCH + r],
                                 gsem.at[r]),
            ))

        out_copies = []
        for r in range(NCH):
            cpa, cpb = gathers[r]
            cpa.wait()
            cpb.wait()
            for k in range(CPR):
                off = (r * CPR + k) * _LANES
                sl = pl.ds(off, _LANES)
                csl = pl.ds(k * _LANES, _LANES)
                zia = jnp.where(xin[sl] == 0.0, 0.0, z[r, csl])
                zib = jnp.where(xin[pl.ds(P + off, _LANES)] == 0.0, 0.0,
                                z[NCH + r, csl])
                logit = zia - zib
                m = jnp.maximum(logit, 0.0)
                t = jnp.exp(-jnp.abs(logit))
                s = t / (t + 2.0)
                s2 = s * s
                log1p_t = 2.0 * s * (1.0 + s2 * ((1.0 / 3.0) + s2 * 0.2))
                lv[sl] = m + log1p_t - xin[pl.ds(2 * P + off, _LANES)] * logit
            out_copies.append(pltpu.async_copy(
                lv.at[pl.ds(r * _CHUNK, _CHUNK)],
                out_hbm.at[pl.ds(base + r * _CHUNK, _CHUNK)], isem))
        for cp in out_copies:
            cp.wait()

    return body


def kernel(x, y, zetas):
    batch = x.shape[0]
    P = batch // _NW
    xf = x.astype(jnp.float32)
    packed = jnp.concatenate(
        [xf[:, 0].reshape(_NW, P), xf[:, 1].reshape(_NW, P),
         y.astype(jnp.float32).reshape(_NW, P)], axis=1).reshape(-1)
    return _build(batch)(packed, zetas.astype(jnp.float32))
